# Initial kernel scaffold; baseline (speedup 1.0000x reference)
import functools

import jax
import jax.numpy as jnp
from jax import lax
from jax.experimental import pallas as pl
from jax.experimental.pallas import tpu as pltpu
from jax.experimental.pallas import tpu_sc as plsc

NC = 2
NS = 16
LW = 128
KROWS = 16
ZROWS = 2048


def _sc_layer_kernel(n, f, r, table, src_r, dst_r, zeros):
    half = n // NC
    cap = ((half + 256 + NS - 1) // NS) * NS
    stripe_z = cap // NS
    ws = half // NS
    rs = r // NS
    nb = rs // KROWS
    assert half % NS == 0 and r % NS == 0 and rs % KROWS == 0
    assert stripe_z >= ZROWS and 2 * ZROWS >= stripe_z

    mesh = plsc.VectorSubcoreMesh(core_axis_name="c", subcore_axis_name="s")

    @functools.partial(
        pl.kernel,
        out_type=jax.ShapeDtypeStruct((n, f), jnp.float32),
        mesh=mesh,
        scratch_types=[
            pltpu.VMEM((ZROWS, f), jnp.float32),
            pltpu.VMEM((KROWS, LW), jnp.int32),
            pltpu.VMEM((KROWS, LW), jnp.int32),
            pltpu.VMEM_SHARED((cap, f), jnp.float32),
        ],
    )
    def seg_kernel(table_hbm, src_hbm, dst_hbm, zeros_hbm, out_hbm,
                   rows_v, src_v, dst_v, acc):
        c = lax.axis_index("c")
        s = lax.axis_index("s")

        pltpu.sync_copy(zeros_hbm, rows_v)
        z0 = s * stripe_z
        pltpu.sync_copy(rows_v, acc.at[pl.ds(z0, ZROWS)])
        pltpu.sync_copy(rows_v, acc.at[pl.ds(z0 + stripe_z - ZROWS, ZROWS)])
        plsc.subcore_barrier()

        row0 = s * rs
        base = c * half

        @pl.loop(0, nb)
        def _(t):
            r0 = row0 + t * KROWS
            pltpu.sync_copy(src_hbm.at[pl.ds(r0, KROWS)], src_v)
            pltpu.sync_copy(dst_hbm.at[pl.ds(r0, KROWS)], dst_v)

            @pl.loop(0, KROWS)
            def _(j):
                @pl.loop(0, LW // 16)
                def _(q):
                    d = dst_v[j, pl.ds(q * 16, 16)]
                    local = d - base
                    ok = (local >= 0) & (local < half)
                    trash = half + (d & 255)
                    dst_v[j, pl.ds(q * 16, 16)] = jnp.where(ok, local, trash)

            @pl.loop(0, KROWS)
            def _(j):
                sl = pl.ds(j * LW, LW)
                pltpu.sync_copy(table_hbm.at[src_v.at[j]], rows_v.at[sl])
                pltpu.sync_copy(rows_v.at[sl], acc.at[dst_v.at[j]], add=True)

        plsc.subcore_barrier()
        pltpu.sync_copy(acc.at[pl.ds(s * ws, ws)],
                        out_hbm.at[pl.ds(base + s * ws, ws)])

    return seg_kernel(table, src_r, dst_r, zeros)


def _sc_deg_kernel(n, r, dst_r, ones, zeros):
    f = 16
    half = n // NC
    cap = ((half + 256 + NS - 1) // NS) * NS
    stripe_z = cap // NS
    ws = half // NS
    rs = r // NS
    nb = rs // KROWS

    mesh = plsc.VectorSubcoreMesh(core_axis_name="c", subcore_axis_name="s")

    @functools.partial(
        pl.kernel,
        out_type=jax.ShapeDtypeStruct((n, f), jnp.float32),
        mesh=mesh,
        scratch_types=[
            pltpu.VMEM((ZROWS, f), jnp.float32),
            pltpu.VMEM((LW, f), jnp.float32),
            pltpu.VMEM((KROWS, LW), jnp.int32),
            pltpu.VMEM_SHARED((cap, f), jnp.float32),
        ],
    )
    def deg_kernel(dst_hbm, ones_hbm, zeros_hbm, out_hbm,
                   zero_v, ones_v, dst_v, acc):
        c = lax.axis_index("c")
        s = lax.axis_index("s")

        pltpu.sync_copy(zeros_hbm, zero_v)
        pltpu.sync_copy(ones_hbm, ones_v)
        z0 = s * stripe_z
        pltpu.sync_copy(zero_v, acc.at[pl.ds(z0, ZROWS)])
        pltpu.sync_copy(zero_v, acc.at[pl.ds(z0 + stripe_z - ZROWS, ZROWS)])
        plsc.subcore_barrier()

        row0 = s * rs
        base = c * half

        @pl.loop(0, nb)
        def _(t):
            r0 = row0 + t * KROWS
            pltpu.sync_copy(dst_hbm.at[pl.ds(r0, KROWS)], dst_v)

            @pl.loop(0, KROWS)
            def _(j):
                @pl.loop(0, LW // 16)
                def _(q):
                    d = dst_v[j, pl.ds(q * 16, 16)]
                    local = d - base
                    ok = (local >= 0) & (local < half)
                    trash = half + (d & 255)
                    dst_v[j, pl.ds(q * 16, 16)] = jnp.where(ok, local, trash)

            @pl.loop(0, KROWS)
            def _(j):
                pltpu.sync_copy(ones_v, acc.at[dst_v.at[j]], add=True)

        plsc.subcore_barrier()
        pltpu.sync_copy(acc.at[pl.ds(s * ws, ws)],
                        out_hbm.at[pl.ds(base + s * ws, ws)])

    return deg_kernel(dst_r, ones, zeros)


def _tc_matmul(x, w):
    n, k = x.shape
    m = w.shape[1]
    bn = 2000
    assert n % bn == 0

    def body(x_ref, w_ref, o_ref):
        o_ref[...] = jnp.dot(x_ref[...], w_ref[...],
                             preferred_element_type=jnp.float32,
                             precision=lax.Precision.HIGHEST)

    return pl.pallas_call(
        body,
        grid=(n // bn,),
        in_specs=[pl.BlockSpec((bn, k), lambda i: (i, 0)),
                  pl.BlockSpec((k, m), lambda i: (0, 0))],
        out_specs=pl.BlockSpec((bn, m), lambda i: (i, 0)),
        out_shape=jax.ShapeDtypeStruct((n, m), jnp.float32),
    )(x, w)


def _tc_scale(h1, deg16):
    n, f = h1.shape
    bn = 2000

    def body(h_ref, d_ref, g_ref, dinv_ref):
        dinv = lax.rsqrt(d_ref[...] + 1.0)
        g_ref[...] = h_ref[...] * dinv[:, :f]
        dinv_ref[...] = dinv

    return pl.pallas_call(
        body,
        grid=(n // bn,),
        in_specs=[pl.BlockSpec((bn, f), lambda i: (i, 0)),
                  pl.BlockSpec((bn, 16), lambda i: (i, 0))],
        out_specs=[pl.BlockSpec((bn, f), lambda i: (i, 0)),
                   pl.BlockSpec((bn, 16), lambda i: (i, 0))],
        out_shape=[jax.ShapeDtypeStruct((n, f), jnp.float32),
                   jax.ShapeDtypeStruct((n, 16), jnp.float32)],
    )(h1, deg16)


def _tc_layer2_in(seg1, g1, dinv16, w2, b1row):
    n, f = seg1.shape
    m = w2.shape[1]
    bn = 2000

    def body(s_ref, g_ref, d_ref, w_ref, b_ref, o_ref):
        dinv = d_ref[...]
        z = dinv[:, :f] * (s_ref[...] + g_ref[...]) + b_ref[...]
        z = jnp.maximum(z, 0.0)
        h2 = jnp.dot(z, w_ref[...], preferred_element_type=jnp.float32,
                     precision=lax.Precision.HIGHEST)
        o_ref[...] = h2 * dinv[:, :1]

    return pl.pallas_call(
        body,
        grid=(n // bn,),
        in_specs=[pl.BlockSpec((bn, f), lambda i: (i, 0)),
                  pl.BlockSpec((bn, f), lambda i: (i, 0)),
                  pl.BlockSpec((bn, 16), lambda i: (i, 0)),
                  pl.BlockSpec((f, m), lambda i: (0, 0)),
                  pl.BlockSpec((1, f), lambda i: (0, 0))],
        out_specs=pl.BlockSpec((bn, m), lambda i: (i, 0)),
        out_shape=jax.ShapeDtypeStruct((n, m), jnp.float32),
    )(seg1, g1, dinv16, w2, b1row)


def _tc_final(seg2, g2, dinv16, b2row):
    n, m = seg2.shape
    bn = 2000

    def body(s_ref, g_ref, d_ref, b_ref, o_ref):
        o = d_ref[:, :1] * (s_ref[...] + g_ref[...]) + b_ref[...]
        mx = jnp.max(o, axis=1, keepdims=True)
        e = jnp.exp(o - mx)
        lse = mx + jnp.log(jnp.sum(e, axis=1, keepdims=True))
        o_ref[...] = o - lse

    return pl.pallas_call(
        body,
        grid=(n // bn,),
        in_specs=[pl.BlockSpec((bn, m), lambda i: (i, 0)),
                  pl.BlockSpec((bn, m), lambda i: (i, 0)),
                  pl.BlockSpec((bn, 16), lambda i: (i, 0)),
                  pl.BlockSpec((1, m), lambda i: (0, 0))],
        out_specs=pl.BlockSpec((bn, m), lambda i: (i, 0)),
        out_shape=jax.ShapeDtypeStruct((n, m), jnp.float32),
    )(seg2, g2, dinv16, b2row)


def kernel(x, edge_index, W1, b1, W2, b2):
    n = x.shape[0]
    e = edge_index.shape[1]
    h = W1.shape[1]
    c = W2.shape[1]

    unit = LW * NS * KROWS
    e_pad = ((e + unit - 1) // unit) * unit
    pad = e_pad - e
    src = edge_index[0]
    dst = edge_index[1]
    if pad:
        src = jnp.concatenate([src, jnp.zeros((pad,), jnp.int32)])
        dst = jnp.concatenate(
            [dst, n + (jnp.arange(pad, dtype=jnp.int32) & 255)])
    r = e_pad // LW
    src_r = src.reshape(r, LW)
    dst_r = dst.reshape(r, LW)

    ones16 = jnp.ones((LW, 16), jnp.float32)
    zeros16 = jnp.zeros((ZROWS, 16), jnp.float32)
    zeros_h = jnp.zeros((ZROWS, h), jnp.float32)
    zeros_c = jnp.zeros((ZROWS, c), jnp.float32)

    deg16 = _sc_deg_kernel(n, r, dst_r, ones16, zeros16)
    h1 = _tc_matmul(x, W1)
    g1, dinv16 = _tc_scale(h1, deg16)
    seg1 = _sc_layer_kernel(n, h, r, g1, src_r, dst_r, zeros_h)
    g2 = _tc_layer2_in(seg1, g1, dinv16, W2, b1.reshape(1, h))
    seg2 = _sc_layer_kernel(n, c, r, g2, src_r, dst_r, zeros_c)
    return _tc_final(seg2, g2, dinv16, b2.reshape(1, c))


# speedup vs baseline: 19.1461x; 19.1461x over previous
import functools

import jax
import jax.numpy as jnp
from jax import lax
from jax.experimental import pallas as pl
from jax.experimental.pallas import tpu as pltpu
from jax.experimental.pallas import tpu_sc as plsc

NC = 2
NS = 16
LW = 128
F = 16
KROWS = 16
ZROWS = 2048

_SC_PARAMS = pltpu.CompilerParams(use_tc_tiling_on_sc=False)


def _sc_geometry(n, r):
    half = n // NC
    cap = ((half + 256 + NS - 1) // NS) * NS
    stripe_z = cap // NS
    ws0 = ((half + NS - 1) // NS + 7) // 8 * 8
    ws_last = half - (NS - 1) * ws0
    rs = r // NS
    nb = rs // KROWS
    assert half % NS == 0 and r % NS == 0 and rs % KROWS == 0
    assert stripe_z >= ZROWS and 2 * ZROWS >= stripe_z
    assert 0 < ws_last <= ws0 and (NS - 1) * ws0 + ws_last == half
    assert cap >= (NS - 1) * ws0 + ws0
    return half, cap, stripe_z, ws0, ws_last, rs, nb


def _clamp_dst(dst_v, base, half):
    @pl.loop(0, KROWS)
    def _(j):
        @pl.loop(0, LW // 16)
        def _(q):
            d = dst_v[j, pl.ds(q * 16, 16)]
            local = d - base
            ok = (local >= 0) & (local < half)
            trash = half + (d & 255)
            dst_v[j, pl.ds(q * 16, 16)] = jnp.where(ok, local, trash)


def _sc_segsum(n, r, table, src_r, dst_r, zeros):
    half, cap, stripe_z, ws0, ws_last, rs, nb = _sc_geometry(n, r)
    mesh = plsc.VectorSubcoreMesh(core_axis_name="c", subcore_axis_name="s")

    @functools.partial(
        pl.kernel,
        out_type=jax.ShapeDtypeStruct((n, F), jnp.float32),
        mesh=mesh,
        scratch_types=[
            pltpu.VMEM((ZROWS, F), jnp.float32),
            pltpu.VMEM((KROWS, LW), jnp.int32),
            pltpu.VMEM((KROWS, LW), jnp.int32),
            pltpu.VMEM_SHARED((cap, F), jnp.float32),
        ],
        compiler_params=_SC_PARAMS,
    )
    def seg_kernel(table_hbm, src_hbm, dst_hbm, zeros_hbm, out_hbm,
                   rows_v, src_v, dst_v, acc):
        c = lax.axis_index("c")
        s = lax.axis_index("s")

        pltpu.sync_copy(zeros_hbm, rows_v)
        z0 = s * stripe_z
        pltpu.sync_copy(rows_v, acc.at[pl.ds(z0, ZROWS)])
        pltpu.sync_copy(rows_v, acc.at[pl.ds(z0 + stripe_z - ZROWS, ZROWS)])
        plsc.subcore_barrier()

        row0 = s * rs
        base = c * half

        @pl.loop(0, nb)
        def _(t):
            r0 = row0 + t * KROWS
            pltpu.sync_copy(src_hbm.at[pl.ds(r0, KROWS)], src_v)
            pltpu.sync_copy(dst_hbm.at[pl.ds(r0, KROWS)], dst_v)
            _clamp_dst(dst_v, base, half)

            @pl.loop(0, KROWS)
            def _(j):
                sl = pl.ds(j * LW, LW)
                pltpu.sync_copy(table_hbm.at[src_v.at[j]], rows_v.at[sl])
                pltpu.sync_copy(rows_v.at[sl], acc.at[dst_v.at[j]], add=True)

        plsc.subcore_barrier()
        w0 = pl.multiple_of(s * ws0, 8)

        @pl.when(s < NS - 1)
        def _():
            pltpu.sync_copy(acc.at[pl.ds(w0, ws0)],
                            out_hbm.at[pl.ds(base + w0, ws0)])

        @pl.when(s == NS - 1)
        def _():
            pltpu.sync_copy(acc.at[pl.ds(w0, ws_last)],
                            out_hbm.at[pl.ds(base + w0, ws_last)])

    return seg_kernel(table, src_r, dst_r, zeros)


def _sc_deg(n, r, dst_r, ones, zeros):
    half, cap, stripe_z, ws0, ws_last, rs, nb = _sc_geometry(n, r)
    mesh = plsc.VectorSubcoreMesh(core_axis_name="c", subcore_axis_name="s")

    @functools.partial(
        pl.kernel,
        out_type=jax.ShapeDtypeStruct((n, F), jnp.float32),
        mesh=mesh,
        scratch_types=[
            pltpu.VMEM((ZROWS, F), jnp.float32),
            pltpu.VMEM((LW, F), jnp.float32),
            pltpu.VMEM((KROWS, LW), jnp.int32),
            pltpu.VMEM_SHARED((cap, F), jnp.float32),
        ],
        compiler_params=_SC_PARAMS,
    )
    def deg_kernel(dst_hbm, ones_hbm, zeros_hbm, out_hbm,
                   zero_v, ones_v, dst_v, acc):
        c = lax.axis_index("c")
        s = lax.axis_index("s")

        pltpu.sync_copy(zeros_hbm, zero_v)
        pltpu.sync_copy(ones_hbm, ones_v)
        z0 = s * stripe_z
        pltpu.sync_copy(zero_v, acc.at[pl.ds(z0, ZROWS)])
        pltpu.sync_copy(zero_v, acc.at[pl.ds(z0 + stripe_z - ZROWS, ZROWS)])
        plsc.subcore_barrier()

        row0 = s * rs
        base = c * half

        @pl.loop(0, nb)
        def _(t):
            r0 = row0 + t * KROWS
            pltpu.sync_copy(dst_hbm.at[pl.ds(r0, KROWS)], dst_v)
            _clamp_dst(dst_v, base, half)

            @pl.loop(0, KROWS)
            def _(j):
                pltpu.sync_copy(ones_v, acc.at[dst_v.at[j]], add=True)

        plsc.subcore_barrier()
        w0 = pl.multiple_of(s * ws0, 8)

        @pl.when(s < NS - 1)
        def _():
            pltpu.sync_copy(acc.at[pl.ds(w0, ws0)],
                            out_hbm.at[pl.ds(base + w0, ws0)])

        @pl.when(s == NS - 1)
        def _():
            pltpu.sync_copy(acc.at[pl.ds(w0, ws_last)],
                            out_hbm.at[pl.ds(base + w0, ws_last)])

    return deg_kernel(dst_r, ones, zeros)


def _tc_matmul(x, w):
    n, k = x.shape
    m = w.shape[1]
    bn = 2000
    assert n % bn == 0

    def body(x_ref, w_ref, o_ref):
        o_ref[...] = jnp.dot(x_ref[...], w_ref[...],
                             preferred_element_type=jnp.float32,
                             precision=lax.Precision.HIGHEST)

    return pl.pallas_call(
        body,
        grid=(n // bn,),
        in_specs=[pl.BlockSpec((bn, k), lambda i: (i, 0)),
                  pl.BlockSpec((k, m), lambda i: (0, 0))],
        out_specs=pl.BlockSpec((bn, m), lambda i: (i, 0)),
        out_shape=jax.ShapeDtypeStruct((n, m), jnp.float32),
    )(x, w)


def _tc_scale(h1, deg):
    n, f = h1.shape
    bn = 2000

    def body(h_ref, d_ref, g_ref, dinv_ref):
        dinv = lax.rsqrt(d_ref[...] + 1.0)
        g_ref[...] = h_ref[...] * dinv
        dinv_ref[...] = dinv

    return pl.pallas_call(
        body,
        grid=(n // bn,),
        in_specs=[pl.BlockSpec((bn, f), lambda i: (i, 0)),
                  pl.BlockSpec((bn, f), lambda i: (i, 0))],
        out_specs=[pl.BlockSpec((bn, f), lambda i: (i, 0)),
                   pl.BlockSpec((bn, f), lambda i: (i, 0))],
        out_shape=[jax.ShapeDtypeStruct((n, f), jnp.float32),
                   jax.ShapeDtypeStruct((n, f), jnp.float32)],
    )(h1, deg)


def _tc_mid(seg1, g1, dinv, b1row):
    n, f = seg1.shape
    bn = 2000

    def body(s_ref, g_ref, d_ref, b_ref, y_ref):
        dinv_b = d_ref[...]
        z = dinv_b * (s_ref[...] + g_ref[...]) + b_ref[...]
        z = jnp.maximum(z, 0.0)
        y_ref[...] = z * dinv_b

    return pl.pallas_call(
        body,
        grid=(n // bn,),
        in_specs=[pl.BlockSpec((bn, f), lambda i: (i, 0)),
                  pl.BlockSpec((bn, f), lambda i: (i, 0)),
                  pl.BlockSpec((bn, f), lambda i: (i, 0)),
                  pl.BlockSpec((1, f), lambda i: (0, 0))],
        out_specs=pl.BlockSpec((bn, f), lambda i: (i, 0)),
        out_shape=jax.ShapeDtypeStruct((n, f), jnp.float32),
    )(seg1, g1, dinv, b1row)


def _tc_final(seg2, y, dinv, w2, b2row):
    n, f = seg2.shape
    m = w2.shape[1]
    bn = 2000

    def body(s_ref, y_ref, d_ref, w_ref, b_ref, o_ref):
        t = s_ref[...] + y_ref[...]
        h2 = jnp.dot(t, w_ref[...], preferred_element_type=jnp.float32,
                     precision=lax.Precision.HIGHEST)
        o = d_ref[:, :1] * h2 + b_ref[...]
        mx = jnp.max(o, axis=1, keepdims=True)
        e = jnp.exp(o - mx)
        lse = mx + jnp.log(jnp.sum(e, axis=1, keepdims=True))
        o_ref[...] = o - lse

    return pl.pallas_call(
        body,
        grid=(n // bn,),
        in_specs=[pl.BlockSpec((bn, f), lambda i: (i, 0)),
                  pl.BlockSpec((bn, f), lambda i: (i, 0)),
                  pl.BlockSpec((bn, f), lambda i: (i, 0)),
                  pl.BlockSpec((f, m), lambda i: (0, 0)),
                  pl.BlockSpec((1, m), lambda i: (0, 0))],
        out_specs=pl.BlockSpec((bn, m), lambda i: (i, 0)),
        out_shape=jax.ShapeDtypeStruct((n, m), jnp.float32),
    )(seg2, y, dinv, w2, b2row)


def kernel(x, edge_index, W1, b1, W2, b2):
    n = x.shape[0]
    e = edge_index.shape[1]
    h = W1.shape[1]
    c = W2.shape[1]
    assert h == F

    unit = LW * NS * KROWS
    e_pad = ((e + unit - 1) // unit) * unit
    pad = e_pad - e
    src = edge_index[0]
    dst = edge_index[1]
    if pad:
        src = jnp.concatenate([src, jnp.zeros((pad,), jnp.int32)])
        dst = jnp.concatenate(
            [dst, n + (jnp.arange(pad, dtype=jnp.int32) & 255)])
    r = e_pad // LW
    src_r = src.reshape(r, LW)
    dst_r = dst.reshape(r, LW)

    ones = jnp.ones((LW, F), jnp.float32)
    zeros = jnp.zeros((ZROWS, F), jnp.float32)

    deg = _sc_deg(n, r, dst_r, ones, zeros)
    h1 = _tc_matmul(x, W1)
    g1, dinv = _tc_scale(h1, deg)
    seg1 = _sc_segsum(n, r, g1, src_r, dst_r, zeros)
    y = _tc_mid(seg1, g1, dinv, b1.reshape(1, h))
    seg2 = _sc_segsum(n, r, y, src_r, dst_r, zeros)
    return _tc_final(seg2, y, dinv, W2, b2.reshape(1, c))


# speedup vs baseline: 26.1798x; 1.3674x over previous
import functools

import jax
import jax.numpy as jnp
from jax import lax
from jax.experimental import pallas as pl
from jax.experimental.pallas import tpu as pltpu
from jax.experimental.pallas import tpu_sc as plsc

NC = 2
NS = 16
LW = 128
F = 16
KROWS = 16
ZROWS = 2048

_SC_PARAMS = pltpu.CompilerParams(use_tc_tiling_on_sc=False)


def _sc_geometry(n, r):
    half = n // NC
    cap = ((half + 256 + NS - 1) // NS) * NS
    stripe_z = cap // NS
    ws0 = ((half + NS - 1) // NS + 7) // 8 * 8
    ws_last = half - (NS - 1) * ws0
    rs = r // NS
    nb = rs // KROWS
    assert half % NS == 0 and r % NS == 0 and rs % KROWS == 0 and nb % 2 == 0
    assert stripe_z >= ZROWS and 2 * ZROWS >= stripe_z
    assert 0 < ws_last <= ws0 and (NS - 1) * ws0 + ws_last == half
    assert cap >= (NS - 1) * ws0 + ws0
    return half, cap, stripe_z, ws0, ws_last, rs, nb


def _clamp_dst(ev, base, half):
    @pl.loop(0, KROWS)
    def _(j):
        @pl.loop(0, LW // 16)
        def _(q):
            d = ev[j, 1, pl.ds(q * 16, 16)]
            local = d - base
            ok = (local >= 0) & (local < half)
            trash = half + (d & 255)
            ev[j, 1, pl.ds(q * 16, 16)] = jnp.where(ok, local, trash)


def _sc_segsum(n, r, table, edges_r, zeros):
    half, cap, stripe_z, ws0, ws_last, rs, nb = _sc_geometry(n, r)
    mesh = plsc.VectorSubcoreMesh(core_axis_name="c", subcore_axis_name="s")

    @functools.partial(
        pl.kernel,
        out_type=jax.ShapeDtypeStruct((n, F), jnp.float32),
        mesh=mesh,
        scratch_types=[
            pltpu.VMEM((ZROWS, F), jnp.float32),
            pltpu.VMEM((ZROWS, F), jnp.float32),
            pltpu.VMEM((KROWS, 2, LW), jnp.int32),
            pltpu.VMEM((KROWS, 2, LW), jnp.int32),
            pltpu.VMEM_SHARED((cap, F), jnp.float32),
            pltpu.SemaphoreType.DMA,
            pltpu.SemaphoreType.DMA,
            pltpu.SemaphoreType.DMA,
            pltpu.SemaphoreType.DMA,
            pltpu.SemaphoreType.DMA,
            pltpu.SemaphoreType.DMA,
        ],
        compiler_params=_SC_PARAMS,
    )
    def seg_kernel(table_hbm, edges_hbm, zeros_hbm, out_hbm,
                   rows0, rows1, ev0, ev1, acc,
                   isem0, isem1, gsem0, gsem1, ssem0, ssem1):
        c = lax.axis_index("c")
        s = lax.axis_index("s")

        pltpu.sync_copy(zeros_hbm, rows0)
        z0 = s * stripe_z
        pltpu.sync_copy(rows0, acc.at[pl.ds(z0, ZROWS)])
        pltpu.sync_copy(rows0, acc.at[pl.ds(z0 + stripe_z - ZROWS, ZROWS)])
        plsc.subcore_barrier()

        row0 = s * rs
        base = c * half
        bufs = ((rows0, ev0, isem0, gsem0, ssem0),
                (rows1, ev1, isem1, gsem1, ssem1))

        def fire_idx(t, b):
            pltpu.async_copy(edges_hbm.at[pl.ds(row0 + t * KROWS, KROWS)],
                             b[1], b[2])

        def wait_idx(t, b):
            pltpu.make_async_copy(
                edges_hbm.at[pl.ds(row0 + t * KROWS, KROWS)], b[1], b[2]).wait()

        def fire_gathers(b):
            rows_v, ev = b[0], b[1]

            @pl.loop(0, KROWS)
            def _(j):
                pltpu.async_copy(table_hbm.at[ev.at[j, 0]],
                                 rows_v.at[pl.ds(j * LW, LW)], b[3])

        def drain_gathers(b):
            rows_v, ev = b[0], b[1]

            @pl.loop(0, KROWS)
            def _(j):
                pltpu.make_async_copy(table_hbm.at[ev.at[j, 0]],
                                      rows_v.at[pl.ds(j * LW, LW)], b[3]).wait()

        def fire_scatters(b):
            rows_v, ev = b[0], b[1]

            @pl.loop(0, KROWS)
            def _(j):
                pltpu.async_copy(rows_v.at[pl.ds(j * LW, LW)],
                                 acc.at[ev.at[j, 1]], b[4], add=True)

        def wait_scatters(b):
            rows_v, ev = b[0], b[1]

            @pl.loop(0, KROWS)
            def _(j):
                pltpu.make_async_copy(rows_v.at[pl.ds(j * LW, LW)],
                                      acc.at[ev.at[j, 1]], b[4]).wait()

        def half_step(t, cur, nxt):
            drain_gathers(cur)
            fire_scatters(cur)

            @pl.when(t + 1 < nb)
            def _():
                wait_idx(t + 1, nxt)
                _clamp_dst(nxt[1], base, half)
                fire_gathers(nxt)

            wait_scatters(cur)

            @pl.when(t + 2 < nb)
            def _():
                fire_idx(t + 2, cur)

        fire_idx(0, bufs[0])
        fire_idx(1, bufs[1])
        wait_idx(0, bufs[0])
        _clamp_dst(bufs[0][1], base, half)
        fire_gathers(bufs[0])

        @pl.loop(0, nb // 2)
        def _(tt):
            half_step(2 * tt, bufs[0], bufs[1])
            half_step(2 * tt + 1, bufs[1], bufs[0])

        plsc.subcore_barrier()
        w0 = pl.multiple_of(s * ws0, 8)

        @pl.when(s < NS - 1)
        def _():
            pltpu.sync_copy(acc.at[pl.ds(w0, ws0)],
                            out_hbm.at[pl.ds(base + w0, ws0)])

        @pl.when(s == NS - 1)
        def _():
            pltpu.sync_copy(acc.at[pl.ds(w0, ws_last)],
                            out_hbm.at[pl.ds(base + w0, ws_last)])

    return seg_kernel(table, edges_r, zeros)


def _sc_deg(n, r, edges_r, ones, zeros):
    half, cap, stripe_z, ws0, ws_last, rs, nb = _sc_geometry(n, r)
    mesh = plsc.VectorSubcoreMesh(core_axis_name="c", subcore_axis_name="s")

    @functools.partial(
        pl.kernel,
        out_type=jax.ShapeDtypeStruct((n, F), jnp.float32),
        mesh=mesh,
        scratch_types=[
            pltpu.VMEM((ZROWS, F), jnp.float32),
            pltpu.VMEM((LW, F), jnp.float32),
            pltpu.VMEM((KROWS, 2, LW), jnp.int32),
            pltpu.VMEM((KROWS, 2, LW), jnp.int32),
            pltpu.VMEM_SHARED((cap, F), jnp.float32),
            pltpu.SemaphoreType.DMA,
            pltpu.SemaphoreType.DMA,
            pltpu.SemaphoreType.DMA,
            pltpu.SemaphoreType.DMA,
        ],
        compiler_params=_SC_PARAMS,
    )
    def deg_kernel(edges_hbm, ones_hbm, zeros_hbm, out_hbm,
                   zero_v, ones_v, ev0, ev1, acc,
                   isem0, isem1, ssem0, ssem1):
        c = lax.axis_index("c")
        s = lax.axis_index("s")

        pltpu.sync_copy(zeros_hbm, zero_v)
        pltpu.sync_copy(ones_hbm, ones_v)
        z0 = s * stripe_z
        pltpu.sync_copy(zero_v, acc.at[pl.ds(z0, ZROWS)])
        pltpu.sync_copy(zero_v, acc.at[pl.ds(z0 + stripe_z - ZROWS, ZROWS)])
        plsc.subcore_barrier()

        row0 = s * rs
        base = c * half
        bufs = ((ev0, isem0, ssem0), (ev1, isem1, ssem1))

        def fire_idx(t, b):
            pltpu.async_copy(edges_hbm.at[pl.ds(row0 + t * KROWS, KROWS)],
                             b[0], b[1])

        def wait_idx(t, b):
            pltpu.make_async_copy(
                edges_hbm.at[pl.ds(row0 + t * KROWS, KROWS)], b[0], b[1]).wait()

        def fire_scatters(b):
            @pl.loop(0, KROWS)
            def _(j):
                pltpu.async_copy(ones_v, acc.at[b[0].at[j, 1]], b[2], add=True)

        def wait_scatters(b):
            @pl.loop(0, KROWS)
            def _(j):
                pltpu.make_async_copy(ones_v, acc.at[b[0].at[j, 1]],
                                      b[2]).wait()

        def half_step(t, cur, nxt):
            wait_idx(t, cur)
            _clamp_dst(cur[0], base, half)
            fire_scatters(cur)
            wait_scatters(cur)

            @pl.when(t + 2 < nb)
            def _():
                fire_idx(t + 2, cur)

        fire_idx(0, bufs[0])
        fire_idx(1, bufs[1])

        @pl.loop(0, nb // 2)
        def _(tt):
            half_step(2 * tt, bufs[0], bufs[1])
            half_step(2 * tt + 1, bufs[1], bufs[0])

        plsc.subcore_barrier()
        w0 = pl.multiple_of(s * ws0, 8)

        @pl.when(s < NS - 1)
        def _():
            pltpu.sync_copy(acc.at[pl.ds(w0, ws0)],
                            out_hbm.at[pl.ds(base + w0, ws0)])

        @pl.when(s == NS - 1)
        def _():
            pltpu.sync_copy(acc.at[pl.ds(w0, ws_last)],
                            out_hbm.at[pl.ds(base + w0, ws_last)])

    return deg_kernel(edges_r, ones, zeros)


def _tc_matmul(x, w):
    n, k = x.shape
    m = w.shape[1]
    bn = 2000
    assert n % bn == 0

    def body(x_ref, w_ref, o_ref):
        o_ref[...] = jnp.dot(x_ref[...], w_ref[...],
                             preferred_element_type=jnp.float32,
                             precision=lax.Precision.HIGHEST)

    return pl.pallas_call(
        body,
        grid=(n // bn,),
        in_specs=[pl.BlockSpec((bn, k), lambda i: (i, 0)),
                  pl.BlockSpec((k, m), lambda i: (0, 0))],
        out_specs=pl.BlockSpec((bn, m), lambda i: (i, 0)),
        out_shape=jax.ShapeDtypeStruct((n, m), jnp.float32),
    )(x, w)


def _tc_scale(h1, deg):
    n, f = h1.shape
    bn = 2000

    def body(h_ref, d_ref, g_ref, dinv_ref):
        dinv = lax.rsqrt(d_ref[...] + 1.0)
        g_ref[...] = h_ref[...] * dinv
        dinv_ref[...] = dinv

    return pl.pallas_call(
        body,
        grid=(n // bn,),
        in_specs=[pl.BlockSpec((bn, f), lambda i: (i, 0)),
                  pl.BlockSpec((bn, f), lambda i: (i, 0))],
        out_specs=[pl.BlockSpec((bn, f), lambda i: (i, 0)),
                   pl.BlockSpec((bn, f), lambda i: (i, 0))],
        out_shape=[jax.ShapeDtypeStruct((n, f), jnp.float32),
                   jax.ShapeDtypeStruct((n, f), jnp.float32)],
    )(h1, deg)


def _tc_mid(seg1, g1, dinv, b1row):
    n, f = seg1.shape
    bn = 2000

    def body(s_ref, g_ref, d_ref, b_ref, y_ref):
        dinv_b = d_ref[...]
        z = dinv_b * (s_ref[...] + g_ref[...]) + b_ref[...]
        z = jnp.maximum(z, 0.0)
        y_ref[...] = z * dinv_b

    return pl.pallas_call(
        body,
        grid=(n // bn,),
        in_specs=[pl.BlockSpec((bn, f), lambda i: (i, 0)),
                  pl.BlockSpec((bn, f), lambda i: (i, 0)),
                  pl.BlockSpec((bn, f), lambda i: (i, 0)),
                  pl.BlockSpec((1, f), lambda i: (0, 0))],
        out_specs=pl.BlockSpec((bn, f), lambda i: (i, 0)),
        out_shape=jax.ShapeDtypeStruct((n, f), jnp.float32),
    )(seg1, g1, dinv, b1row)


def _tc_final(seg2, y, dinv, w2, b2row):
    n, f = seg2.shape
    m = w2.shape[1]
    bn = 2000

    def body(s_ref, y_ref, d_ref, w_ref, b_ref, o_ref):
        t = s_ref[...] + y_ref[...]
        h2 = jnp.dot(t, w_ref[...], preferred_element_type=jnp.float32,
                     precision=lax.Precision.HIGHEST)
        o = d_ref[:, :1] * h2 + b_ref[...]
        mx = jnp.max(o, axis=1, keepdims=True)
        e = jnp.exp(o - mx)
        lse = mx + jnp.log(jnp.sum(e, axis=1, keepdims=True))
        o_ref[...] = o - lse

    return pl.pallas_call(
        body,
        grid=(n // bn,),
        in_specs=[pl.BlockSpec((bn, f), lambda i: (i, 0)),
                  pl.BlockSpec((bn, f), lambda i: (i, 0)),
                  pl.BlockSpec((bn, f), lambda i: (i, 0)),
                  pl.BlockSpec((f, m), lambda i: (0, 0)),
                  pl.BlockSpec((1, m), lambda i: (0, 0))],
        out_specs=pl.BlockSpec((bn, m), lambda i: (i, 0)),
        out_shape=jax.ShapeDtypeStruct((n, m), jnp.float32),
    )(seg2, y, dinv, w2, b2row)


def kernel(x, edge_index, W1, b1, W2, b2):
    n = x.shape[0]
    e = edge_index.shape[1]
    h = W1.shape[1]
    c = W2.shape[1]
    assert h == F

    unit = LW * NS * KROWS * 2
    e_pad = ((e + unit - 1) // unit) * unit
    pad = e_pad - e
    src = edge_index[0]
    dst = edge_index[1]
    if pad:
        src = jnp.concatenate([src, jnp.zeros((pad,), jnp.int32)])
        dst = jnp.concatenate(
            [dst, n + (jnp.arange(pad, dtype=jnp.int32) & 255)])
    r = e_pad // LW
    edges_r = jnp.stack([src.reshape(r, LW), dst.reshape(r, LW)], axis=1)

    ones = jnp.ones((LW, F), jnp.float32)
    zeros = jnp.zeros((ZROWS, F), jnp.float32)

    deg = _sc_deg(n, r, edges_r, ones, zeros)
    h1 = _tc_matmul(x, W1)
    g1, dinv = _tc_scale(h1, deg)
    seg1 = _sc_segsum(n, r, g1, edges_r, zeros)
    y = _tc_mid(seg1, g1, dinv, b1.reshape(1, h))
    seg2 = _sc_segsum(n, r, y, edges_r, zeros)
    return _tc_final(seg2, y, dinv, W2, b2.reshape(1, c))


# speedup vs baseline: 40.6758x; 1.5537x over previous
import functools

import jax
import jax.numpy as jnp
from jax import lax
from jax.experimental import pallas as pl
from jax.experimental.pallas import tpu as pltpu
from jax.experimental.pallas import tpu_sc as plsc

NC = 2
NS = 16
LW = 128
F = 16
KROWS = 4
ZROWS = 512

_SC_PARAMS = pltpu.CompilerParams(use_tc_tiling_on_sc=False)


def _sc_geometry(n, r):
    cap = ((n + 256 + NS - 1) // NS) * NS
    stripe_z = cap // NS
    nzc = (stripe_z + ZROWS - 1) // ZROWS
    ws0 = ((n + NS - 1) // NS + 7) // 8 * 8
    ws_last = n - (NS - 1) * ws0
    rs = r // (NC * NS)
    nb = rs // KROWS
    assert n % NS == 0 and r % (NC * NS) == 0 and rs % KROWS == 0
    assert nb % 2 == 0 and stripe_z >= ZROWS
    assert 0 < ws_last <= ws0 and (NS - 1) * ws0 + ws_last == n
    assert cap >= (NS - 1) * ws0 + ws0
    return cap, stripe_z, nzc, ws0, ws_last, rs, nb


def _zero_acc(acc, zsrc, s, stripe_z, nzc):
    z0 = s * stripe_z

    @pl.loop(0, nzc - 1)
    def _(i):
        pltpu.sync_copy(zsrc, acc.at[pl.ds(z0 + i * ZROWS, ZROWS)])

    pltpu.sync_copy(zsrc, acc.at[pl.ds(z0 + stripe_z - ZROWS, ZROWS)])


def _readout(acc, out_hbm, s, ws0, ws_last):
    w0 = pl.multiple_of(s * ws0, 8)

    @pl.when(s < NS - 1)
    def _():
        pltpu.sync_copy(acc.at[pl.ds(w0, ws0)], out_hbm.at[pl.ds(w0, ws0)])

    @pl.when(s == NS - 1)
    def _():
        pltpu.sync_copy(acc.at[pl.ds(w0, ws_last)],
                        out_hbm.at[pl.ds(w0, ws_last)])


def _sc_segsum(n, r, table, edges_r, zeros):
    cap, stripe_z, nzc, ws0, ws_last, rs, nb = _sc_geometry(n, r)
    mesh = plsc.VectorSubcoreMesh(core_axis_name="c", subcore_axis_name="s")
    out_t = jax.ShapeDtypeStruct((n, F), jnp.float32)

    @functools.partial(
        pl.kernel,
        out_type=[out_t, out_t],
        mesh=mesh,
        scratch_types=[
            pltpu.VMEM((KROWS * LW, F), jnp.float32),
            pltpu.VMEM((KROWS * LW, F), jnp.float32),
            pltpu.VMEM((2 * KROWS, LW), jnp.int32),
            pltpu.VMEM((2 * KROWS, LW), jnp.int32),
            pltpu.VMEM_SHARED((cap, F), jnp.float32),
            pltpu.SemaphoreType.DMA,
            pltpu.SemaphoreType.DMA,
            pltpu.SemaphoreType.DMA,
            pltpu.SemaphoreType.DMA,
            pltpu.SemaphoreType.DMA,
            pltpu.SemaphoreType.DMA,
        ],
        compiler_params=_SC_PARAMS,
    )
    def seg_kernel(table_hbm, edges_hbm, zeros_hbm, out0_hbm, out1_hbm,
                   rows0, rows1, ev0, ev1, acc,
                   isem0, isem1, gsem0, gsem1, ssem0, ssem1):
        c = lax.axis_index("c")
        s = lax.axis_index("s")

        pltpu.sync_copy(zeros_hbm, rows0)
        _zero_acc(acc, rows0, s, stripe_z, nzc)
        plsc.subcore_barrier()

        row0 = (c * NS + s) * rs
        bufs = ((rows0, ev0, isem0, gsem0, ssem0),
                (rows1, ev1, isem1, gsem1, ssem1))

        def fire_idx(t, b):
            pltpu.async_copy(
                edges_hbm.at[pl.ds(2 * (row0 + t * KROWS), 2 * KROWS)],
                b[1], b[2])

        def wait_idx(t, b):
            pltpu.make_async_copy(
                edges_hbm.at[pl.ds(2 * (row0 + t * KROWS), 2 * KROWS)],
                b[1], b[2]).wait()

        def fire_gathers(b):
            @pl.loop(0, KROWS)
            def _(j):
                pltpu.async_copy(table_hbm.at[b[1].at[2 * j]],
                                 b[0].at[pl.ds(j * LW, LW)], b[3])

        def drain_gathers(b):
            @pl.loop(0, KROWS)
            def _(j):
                pltpu.make_async_copy(table_hbm.at[b[1].at[2 * j]],
                                      b[0].at[pl.ds(j * LW, LW)], b[3]).wait()

        def fire_scatters(b):
            @pl.loop(0, KROWS)
            def _(j):
                pltpu.async_copy(b[0].at[pl.ds(j * LW, LW)],
                                 acc.at[b[1].at[2 * j + 1]], b[4], add=True)

        def wait_scatters(b):
            @pl.loop(0, KROWS)
            def _(j):
                pltpu.make_async_copy(b[0].at[pl.ds(j * LW, LW)],
                                      acc.at[b[1].at[2 * j + 1]], b[4]).wait()

        def half_step(t, cur, nxt):
            drain_gathers(cur)
            fire_scatters(cur)

            @pl.when(t + 1 < nb)
            def _():
                wait_idx(t + 1, nxt)
                fire_gathers(nxt)

            wait_scatters(cur)

            @pl.when(t + 2 < nb)
            def _():
                fire_idx(t + 2, cur)

        fire_idx(0, bufs[0])
        fire_idx(1, bufs[1])
        wait_idx(0, bufs[0])
        fire_gathers(bufs[0])

        @pl.loop(0, nb // 2)
        def _(tt):
            half_step(2 * tt, bufs[0], bufs[1])
            half_step(2 * tt + 1, bufs[1], bufs[0])

        plsc.subcore_barrier()

        @pl.when(c == 0)
        def _():
            _readout(acc, out0_hbm, s, ws0, ws_last)

        @pl.when(c == 1)
        def _():
            _readout(acc, out1_hbm, s, ws0, ws_last)

    return seg_kernel(table, edges_r, zeros)


def _sc_deg(n, r, edges_r, ones, zeros):
    cap, stripe_z, nzc, ws0, ws_last, rs, nb = _sc_geometry(n, r)
    mesh = plsc.VectorSubcoreMesh(core_axis_name="c", subcore_axis_name="s")
    out_t = jax.ShapeDtypeStruct((n, F), jnp.float32)

    @functools.partial(
        pl.kernel,
        out_type=[out_t, out_t],
        mesh=mesh,
        scratch_types=[
            pltpu.VMEM((ZROWS, F), jnp.float32),
            pltpu.VMEM((LW, F), jnp.float32),
            pltpu.VMEM((2 * KROWS, LW), jnp.int32),
            pltpu.VMEM((2 * KROWS, LW), jnp.int32),
            pltpu.VMEM_SHARED((cap, F), jnp.float32),
            pltpu.SemaphoreType.DMA,
            pltpu.SemaphoreType.DMA,
            pltpu.SemaphoreType.DMA,
            pltpu.SemaphoreType.DMA,
        ],
        compiler_params=_SC_PARAMS,
    )
    def deg_kernel(edges_hbm, ones_hbm, zeros_hbm, out0_hbm, out1_hbm,
                   zero_v, ones_v, ev0, ev1, acc,
                   isem0, isem1, ssem0, ssem1):
        c = lax.axis_index("c")
        s = lax.axis_index("s")

        pltpu.sync_copy(zeros_hbm, zero_v)
        pltpu.sync_copy(ones_hbm, ones_v)
        _zero_acc(acc, zero_v, s, stripe_z, nzc)
        plsc.subcore_barrier()

        row0 = (c * NS + s) * rs
        bufs = ((ev0, isem0, ssem0), (ev1, isem1, ssem1))

        def fire_idx(t, b):
            pltpu.async_copy(
                edges_hbm.at[pl.ds(2 * (row0 + t * KROWS), 2 * KROWS)],
                b[0], b[1])

        def wait_idx(t, b):
            pltpu.make_async_copy(
                edges_hbm.at[pl.ds(2 * (row0 + t * KROWS), 2 * KROWS)],
                b[0], b[1]).wait()

        def fire_scatters(b):
            @pl.loop(0, KROWS)
            def _(j):
                pltpu.async_copy(ones_v, acc.at[b[0].at[2 * j + 1]],
                                 b[2], add=True)

        def wait_scatters(b):
            @pl.loop(0, KROWS)
            def _(j):
                pltpu.make_async_copy(ones_v, acc.at[b[0].at[2 * j + 1]],
                                      b[2]).wait()

        def half_step(t, cur):
            wait_idx(t, cur)
            fire_scatters(cur)
            wait_scatters(cur)

            @pl.when(t + 2 < nb)
            def _():
                fire_idx(t + 2, cur)

        fire_idx(0, bufs[0])
        fire_idx(1, bufs[1])

        @pl.loop(0, nb // 2)
        def _(tt):
            half_step(2 * tt, bufs[0])
            half_step(2 * tt + 1, bufs[1])

        plsc.subcore_barrier()

        @pl.when(c == 0)
        def _():
            _readout(acc, out0_hbm, s, ws0, ws_last)

        @pl.when(c == 1)
        def _():
            _readout(acc, out1_hbm, s, ws0, ws_last)

    return deg_kernel(edges_r, ones, zeros)


def _tc_matmul(x, w):
    n, k = x.shape
    m = w.shape[1]
    bn = 2000
    assert n % bn == 0

    def body(x_ref, w_ref, o_ref):
        o_ref[...] = jnp.dot(x_ref[...], w_ref[...],
                             preferred_element_type=jnp.float32,
                             precision=lax.Precision.HIGHEST)

    return pl.pallas_call(
        body,
        grid=(n // bn,),
        in_specs=[pl.BlockSpec((bn, k), lambda i: (i, 0)),
                  pl.BlockSpec((k, m), lambda i: (0, 0))],
        out_specs=pl.BlockSpec((bn, m), lambda i: (i, 0)),
        out_shape=jax.ShapeDtypeStruct((n, m), jnp.float32),
    )(x, w)


def _tc_scale(h1, deg_a, deg_b):
    n, f = h1.shape
    bn = 2000
    spec = pl.BlockSpec((bn, f), lambda i: (i, 0))

    def body(h_ref, da_ref, db_ref, g_ref, dinv_ref):
        dinv = lax.rsqrt(da_ref[...] + db_ref[...] + 1.0)
        g_ref[...] = h_ref[...] * dinv
        dinv_ref[...] = dinv

    return pl.pallas_call(
        body,
        grid=(n // bn,),
        in_specs=[spec, spec, spec],
        out_specs=[spec, spec],
        out_shape=[jax.ShapeDtypeStruct((n, f), jnp.float32),
                   jax.ShapeDtypeStruct((n, f), jnp.float32)],
    )(h1, deg_a, deg_b)


def _tc_mid(seg_a, seg_b, g1, dinv, b1row):
    n, f = seg_a.shape
    bn = 2000
    spec = pl.BlockSpec((bn, f), lambda i: (i, 0))

    def body(sa_ref, sb_ref, g_ref, d_ref, b_ref, y_ref):
        dinv_b = d_ref[...]
        z = dinv_b * (sa_ref[...] + sb_ref[...] + g_ref[...]) + b_ref[...]
        z = jnp.maximum(z, 0.0)
        y_ref[...] = z * dinv_b

    return pl.pallas_call(
        body,
        grid=(n // bn,),
        in_specs=[spec, spec, spec, spec,
                  pl.BlockSpec((1, f), lambda i: (0, 0))],
        out_specs=spec,
        out_shape=jax.ShapeDtypeStruct((n, f), jnp.float32),
    )(seg_a, seg_b, g1, dinv, b1row)


def _tc_final(seg_a, seg_b, y, dinv, w2, b2row):
    n, f = seg_a.shape
    m = w2.shape[1]
    bn = 2000
    spec = pl.BlockSpec((bn, f), lambda i: (i, 0))

    def body(sa_ref, sb_ref, y_ref, d_ref, w_ref, b_ref, o_ref):
        t = sa_ref[...] + sb_ref[...] + y_ref[...]
        h2 = jnp.dot(t, w_ref[...], preferred_element_type=jnp.float32,
                     precision=lax.Precision.HIGHEST)
        o = d_ref[:, :1] * h2 + b_ref[...]
        mx = jnp.max(o, axis=1, keepdims=True)
        e = jnp.exp(o - mx)
        lse = mx + jnp.log(jnp.sum(e, axis=1, keepdims=True))
        o_ref[...] = o - lse

    return pl.pallas_call(
        body,
        grid=(n // bn,),
        in_specs=[spec, spec, spec, spec,
                  pl.BlockSpec((f, m), lambda i: (0, 0)),
                  pl.BlockSpec((1, m), lambda i: (0, 0))],
        out_specs=pl.BlockSpec((bn, m), lambda i: (i, 0)),
        out_shape=jax.ShapeDtypeStruct((n, m), jnp.float32),
    )(seg_a, seg_b, y, dinv, w2, b2row)


def kernel(x, edge_index, W1, b1, W2, b2):
    n = x.shape[0]
    e = edge_index.shape[1]
    h = W1.shape[1]
    c = W2.shape[1]
    assert h == F

    unit = LW * NC * NS * KROWS * 2
    e_pad = ((e + unit - 1) // unit) * unit
    pad = e_pad - e
    src = edge_index[0]
    dst = edge_index[1]
    if pad:
        src = jnp.concatenate([src, jnp.zeros((pad,), jnp.int32)])
        dst = jnp.concatenate(
            [dst, n + (jnp.arange(pad, dtype=jnp.int32) & 255)])
    r = e_pad // LW
    edges_r = jnp.stack([src.reshape(r, LW), dst.reshape(r, LW)],
                        axis=1).reshape(2 * r, LW)

    ones = jnp.ones((LW, F), jnp.float32)
    zeros = jnp.zeros((ZROWS, F), jnp.float32)

    deg_a, deg_b = _sc_deg(n, r, edges_r, ones, zeros)
    h1 = _tc_matmul(x, W1)
    g1, dinv = _tc_scale(h1, deg_a, deg_b)
    seg1a, seg1b = _sc_segsum(n, r, g1, edges_r, zeros)
    y = _tc_mid(seg1a, seg1b, g1, dinv, b1.reshape(1, h))
    seg2a, seg2b = _sc_segsum(n, r, y, edges_r, zeros)
    return _tc_final(seg2a, seg2b, y, dinv, W2, b2.reshape(1, c))


# speedup vs baseline: 55.7924x; 1.3716x over previous
import functools

import jax
import jax.numpy as jnp
from jax import lax
from jax.experimental import pallas as pl
from jax.experimental.pallas import tpu as pltpu
from jax.experimental.pallas import tpu_sc as plsc

NC = 2
NS = 16
LW = 128
F = 16
KROWS = 4
ZROWS = 512

_SC_PARAMS = pltpu.CompilerParams(use_tc_tiling_on_sc=False)


def _sc_geometry(n, r):
    cap = ((n + 256 + NS - 1) // NS) * NS
    stripe_z = cap // NS
    nzc = (stripe_z + ZROWS - 1) // ZROWS
    ws0 = ((n + NS - 1) // NS + 7) // 8 * 8
    ws_last = n - (NS - 1) * ws0
    rs = r // (NC * NS)
    nb = rs // KROWS
    assert n % NS == 0 and r % (NC * NS) == 0 and rs % KROWS == 0
    assert nb % 2 == 0 and stripe_z >= ZROWS
    assert 0 < ws_last <= ws0 and (NS - 1) * ws0 + ws_last == n
    assert cap >= (NS - 1) * ws0 + ws0
    return cap, stripe_z, nzc, ws0, ws_last, rs, nb


def _zero_acc(acc, zsrc, s, stripe_z, nzc):
    z0 = s * stripe_z

    @pl.loop(0, nzc - 1)
    def _(i):
        pltpu.sync_copy(zsrc, acc.at[pl.ds(z0 + i * ZROWS, ZROWS)])

    pltpu.sync_copy(zsrc, acc.at[pl.ds(z0 + stripe_z - ZROWS, ZROWS)])


def _readout(acc, out_hbm, s, ws0, ws_last):
    w0 = pl.multiple_of(s * ws0, 8)

    @pl.when(s < NS - 1)
    def _():
        pltpu.sync_copy(acc.at[pl.ds(w0, ws0)], out_hbm.at[pl.ds(w0, ws0)])

    @pl.when(s == NS - 1)
    def _():
        pltpu.sync_copy(acc.at[pl.ds(w0, ws_last)],
                        out_hbm.at[pl.ds(w0, ws_last)])


def _sc_segsum(n, r, table, edges_r, zeros):
    cap, stripe_z, nzc, ws0, ws_last, rs, nb = _sc_geometry(n, r)
    mesh = plsc.VectorSubcoreMesh(core_axis_name="c", subcore_axis_name="s")
    out_t = jax.ShapeDtypeStruct((n, F), jnp.float32)

    @functools.partial(
        pl.kernel,
        out_type=[out_t, out_t],
        mesh=mesh,
        scratch_types=[
            pltpu.VMEM((KROWS * LW, F), jnp.float32),
            pltpu.VMEM((KROWS * LW, F), jnp.float32),
            pltpu.VMEM((2 * KROWS, LW), jnp.int32),
            pltpu.VMEM((2 * KROWS, LW), jnp.int32),
            pltpu.VMEM_SHARED((cap, F), jnp.float32),
            pltpu.SemaphoreType.DMA,
            pltpu.SemaphoreType.DMA,
            pltpu.SemaphoreType.DMA,
            pltpu.SemaphoreType.DMA,
            pltpu.SemaphoreType.DMA,
            pltpu.SemaphoreType.DMA,
        ],
        compiler_params=_SC_PARAMS,
    )
    def seg_kernel(table_hbm, edges_hbm, zeros_hbm, out0_hbm, out1_hbm,
                   rows0, rows1, ev0, ev1, acc,
                   isem0, isem1, gsem0, gsem1, ssem0, ssem1):
        c = lax.axis_index("c")
        s = lax.axis_index("s")

        pltpu.sync_copy(zeros_hbm, rows0)
        _zero_acc(acc, rows0, s, stripe_z, nzc)
        plsc.subcore_barrier()

        row0 = (c * NS + s) * rs
        bufs = ((rows0, ev0, isem0, gsem0, ssem0),
                (rows1, ev1, isem1, gsem1, ssem1))

        def fire_idx(t, b):
            pltpu.async_copy(
                edges_hbm.at[pl.ds(2 * (row0 + t * KROWS), 2 * KROWS)],
                b[1], b[2])

        def wait_idx(t, b):
            pltpu.make_async_copy(
                edges_hbm.at[pl.ds(2 * (row0 + t * KROWS), 2 * KROWS)],
                b[1], b[2]).wait()

        def fire_gathers(b):
            @pl.loop(0, KROWS)
            def _(j):
                pltpu.async_copy(table_hbm.at[b[1].at[2 * j]],
                                 b[0].at[pl.ds(j * LW, LW)], b[3])

        def drain_gathers(b):
            @pl.loop(0, KROWS)
            def _(j):
                pltpu.make_async_copy(table_hbm.at[b[1].at[2 * j]],
                                      b[0].at[pl.ds(j * LW, LW)], b[3]).wait()

        def fire_scatters(b):
            @pl.loop(0, KROWS)
            def _(j):
                pltpu.async_copy(b[0].at[pl.ds(j * LW, LW)],
                                 acc.at[b[1].at[2 * j + 1]], b[4], add=True)

        def wait_scatters(b):
            @pl.loop(0, KROWS)
            def _(j):
                pltpu.make_async_copy(b[0].at[pl.ds(j * LW, LW)],
                                      acc.at[b[1].at[2 * j + 1]], b[4]).wait()

        def half_step(t, cur, nxt):
            drain_gathers(cur)
            fire_scatters(cur)

            @pl.when(t + 1 < nb)
            def _():
                wait_idx(t + 1, nxt)
                fire_gathers(nxt)

            wait_scatters(cur)

            @pl.when(t + 2 < nb)
            def _():
                fire_idx(t + 2, cur)

        fire_idx(0, bufs[0])
        fire_idx(1, bufs[1])
        wait_idx(0, bufs[0])
        fire_gathers(bufs[0])

        @pl.loop(0, nb // 2)
        def _(tt):
            half_step(2 * tt, bufs[0], bufs[1])
            half_step(2 * tt + 1, bufs[1], bufs[0])

        plsc.subcore_barrier()

        @pl.when(c == 0)
        def _():
            _readout(acc, out0_hbm, s, ws0, ws_last)

        @pl.when(c == 1)
        def _():
            _readout(acc, out1_hbm, s, ws0, ws_last)

    return seg_kernel(table, edges_r, zeros)


def _sc_deg(n, r, edges_r, ones, zeros):
    cap, stripe_z, nzc, ws0, ws_last, rs, nb = _sc_geometry(n, r)
    mesh = plsc.VectorSubcoreMesh(core_axis_name="c", subcore_axis_name="s")
    out_t = jax.ShapeDtypeStruct((n, F), jnp.float32)

    @functools.partial(
        pl.kernel,
        out_type=[out_t, out_t],
        mesh=mesh,
        scratch_types=[
            pltpu.VMEM((ZROWS, F), jnp.float32),
            pltpu.VMEM((LW, F), jnp.float32),
            pltpu.VMEM((2 * KROWS, LW), jnp.int32),
            pltpu.VMEM((2 * KROWS, LW), jnp.int32),
            pltpu.VMEM_SHARED((cap, F), jnp.float32),
            pltpu.SemaphoreType.DMA,
            pltpu.SemaphoreType.DMA,
            pltpu.SemaphoreType.DMA,
            pltpu.SemaphoreType.DMA,
        ],
        compiler_params=_SC_PARAMS,
    )
    def deg_kernel(edges_hbm, ones_hbm, zeros_hbm, out0_hbm, out1_hbm,
                   zero_v, ones_v, ev0, ev1, acc,
                   isem0, isem1, ssem0, ssem1):
        c = lax.axis_index("c")
        s = lax.axis_index("s")

        pltpu.sync_copy(zeros_hbm, zero_v)
        pltpu.sync_copy(ones_hbm, ones_v)
        _zero_acc(acc, zero_v, s, stripe_z, nzc)
        plsc.subcore_barrier()

        row0 = (c * NS + s) * rs
        bufs = ((ev0, isem0, ssem0), (ev1, isem1, ssem1))

        def fire_idx(t, b):
            pltpu.async_copy(
                edges_hbm.at[pl.ds(2 * (row0 + t * KROWS), 2 * KROWS)],
                b[0], b[1])

        def wait_idx(t, b):
            pltpu.make_async_copy(
                edges_hbm.at[pl.ds(2 * (row0 + t * KROWS), 2 * KROWS)],
                b[0], b[1]).wait()

        def fire_scatters(b):
            @pl.loop(0, KROWS)
            def _(j):
                pltpu.async_copy(ones_v, acc.at[b[0].at[2 * j + 1]],
                                 b[2], add=True)

        def wait_scatters(b):
            @pl.loop(0, KROWS)
            def _(j):
                pltpu.make_async_copy(ones_v, acc.at[b[0].at[2 * j + 1]],
                                      b[2]).wait()

        def half_step(t, cur):
            wait_idx(t, cur)
            fire_scatters(cur)
            wait_scatters(cur)

            @pl.when(t + 2 < nb)
            def _():
                fire_idx(t + 2, cur)

        fire_idx(0, bufs[0])
        fire_idx(1, bufs[1])

        @pl.loop(0, nb // 2)
        def _(tt):
            half_step(2 * tt, bufs[0])
            half_step(2 * tt + 1, bufs[1])

        plsc.subcore_barrier()

        @pl.when(c == 0)
        def _():
            _readout(acc, out0_hbm, s, ws0, ws_last)

        @pl.when(c == 1)
        def _():
            _readout(acc, out1_hbm, s, ws0, ws_last)

    return deg_kernel(edges_r, ones, zeros)


G = 25
BP = 500


def _tc_matmul(x, w):
    n, k = x.shape
    m = w.shape[1]
    bn = n // G

    def body(x_ref, w_ref, o_ref):
        o_ref[...] = jnp.dot(x_ref[...], w_ref[...],
                             preferred_element_type=jnp.float32,
                             precision=lax.Precision.HIGHEST)

    return pl.pallas_call(
        body,
        grid=(G,),
        in_specs=[pl.BlockSpec((bn, k), lambda i: (i, 0)),
                  pl.BlockSpec((k, m), lambda i: (0, 0))],
        out_specs=pl.BlockSpec((bn, m), lambda i: (i, 0)),
        out_shape=jax.ShapeDtypeStruct((n, m), jnp.float32),
    )(x, w)


_P3SPEC = pl.BlockSpec((1, BP, 128), lambda i: (i, 0, 0))


def _tc_scale(h1p, deg_a, deg_b):
    def body(h_ref, da_ref, db_ref, g_ref, dinv_ref):
        dinv = lax.rsqrt(da_ref[...] + db_ref[...] + 1.0)
        g_ref[...] = h_ref[...] * dinv
        dinv_ref[...] = dinv

    out_t = jax.ShapeDtypeStruct((G, BP, 128), jnp.float32)
    return pl.pallas_call(
        body,
        grid=(G,),
        in_specs=[_P3SPEC, _P3SPEC, _P3SPEC],
        out_specs=[_P3SPEC, _P3SPEC],
        out_shape=[out_t, out_t],
    )(h1p, deg_a, deg_b)


def _tc_mid(seg_a, seg_b, g1p, dinvp, b1tile):
    def body(sa_ref, sb_ref, g_ref, d_ref, b_ref, y_ref):
        dinv_b = d_ref[...]
        z = dinv_b * (sa_ref[...] + sb_ref[...] + g_ref[...]) + b_ref[...]
        z = jnp.maximum(z, 0.0)
        y_ref[...] = z * dinv_b

    return pl.pallas_call(
        body,
        grid=(G,),
        in_specs=[_P3SPEC, _P3SPEC, _P3SPEC, _P3SPEC,
                  pl.BlockSpec((1, 1, 128), lambda i: (0, 0, 0))],
        out_specs=_P3SPEC,
        out_shape=jax.ShapeDtypeStruct((G, BP, 128), jnp.float32),
    )(seg_a, seg_b, g1p, dinvp, b1tile)


def _tc_out_packed(seg_a, seg_b, yp, dinvp, w2big, sel, b2tile):
    mp = w2big.shape[1]

    def body(sa_ref, sb_ref, y_ref, d_ref, w_ref, s_ref, b_ref, o_ref):
        t = sa_ref[0] + sb_ref[0] + y_ref[0]
        h2 = jnp.dot(t, w_ref[...], preferred_element_type=jnp.float32,
                     precision=lax.Precision.HIGHEST)
        dsc = jnp.dot(d_ref[0], s_ref[...], preferred_element_type=jnp.float32,
                      precision=lax.Precision.HIGHEST)
        o_ref[0] = dsc * h2 + b_ref[0]

    return pl.pallas_call(
        body,
        grid=(G,),
        in_specs=[_P3SPEC, _P3SPEC, _P3SPEC, _P3SPEC,
                  pl.BlockSpec((128, mp), lambda i: (0, 0)),
                  pl.BlockSpec((128, mp), lambda i: (0, 0)),
                  pl.BlockSpec((1, 1, mp), lambda i: (0, 0, 0))],
        out_specs=pl.BlockSpec((1, BP, mp), lambda i: (i, 0, 0)),
        out_shape=jax.ShapeDtypeStruct((G, BP, mp), jnp.float32),
    )(seg_a, seg_b, yp, dinvp, w2big, sel, b2tile)


def _tc_log_softmax(o):
    n, m = o.shape
    bn = n // G

    def body(o_ref, out_ref):
        ob = o_ref[...]
        mx = jnp.max(ob, axis=1, keepdims=True)
        e = jnp.exp(ob - mx)
        lse = mx + jnp.log(jnp.sum(e, axis=1, keepdims=True))
        out_ref[...] = ob - lse

    spec = pl.BlockSpec((bn, m), lambda i: (i, 0))
    return pl.pallas_call(
        body,
        grid=(G,),
        in_specs=[spec],
        out_specs=spec,
        out_shape=jax.ShapeDtypeStruct((n, m), jnp.float32),
    )(o)


def kernel(x, edge_index, W1, b1, W2, b2):
    n = x.shape[0]
    e = edge_index.shape[1]
    h = W1.shape[1]
    c = W2.shape[1]
    assert h == F

    unit = LW * NC * NS * KROWS * 2
    e_pad = ((e + unit - 1) // unit) * unit
    pad = e_pad - e
    src = edge_index[0]
    dst = edge_index[1]
    if pad:
        src = jnp.concatenate([src, jnp.zeros((pad,), jnp.int32)])
        dst = jnp.concatenate(
            [dst, n + (jnp.arange(pad, dtype=jnp.int32) & 255)])
    r = e_pad // LW
    edges_r = jnp.stack([src.reshape(r, LW), dst.reshape(r, LW)],
                        axis=1).reshape(2 * r, LW)

    ones = jnp.ones((LW, F), jnp.float32)
    zeros = jnp.zeros((ZROWS, F), jnp.float32)

    def pk(a):
        return a.reshape(G, BP, 8 * F)

    def un(ap):
        return ap.reshape(n, F)

    deg_a, deg_b = _sc_deg(n, r, edges_r, ones, zeros)
    h1 = _tc_matmul(x, W1)
    g1p, dinvp = _tc_scale(h1.reshape(G, BP, 8 * h), pk(deg_a), pk(deg_b))
    seg1a, seg1b = _sc_segsum(n, r, un(g1p), edges_r, zeros)
    yp = _tc_mid(pk(seg1a), pk(seg1b), g1p, dinvp,
                 jnp.tile(b1, 8).reshape(1, 1, 8 * h))
    seg2a, seg2b = _sc_segsum(n, r, un(yp), edges_r, zeros)
    w2big = jnp.kron(jnp.eye(8, dtype=jnp.float32), W2)
    sel = jnp.kron(jnp.eye(8, dtype=jnp.float32),
                   jnp.zeros((h, c), jnp.float32).at[0, :].set(1.0))
    b2tile = jnp.tile(b2, 8).reshape(1, 1, 8 * c)
    o_p = _tc_out_packed(pk(seg2a), pk(seg2b), yp, dinvp, w2big, sel, b2tile)
    return _tc_log_softmax(o_p.reshape(n, c))


# speedup vs baseline: 59.5696x; 1.0677x over previous
import functools

import jax
import jax.numpy as jnp
from jax import lax
from jax.experimental import pallas as pl
from jax.experimental.pallas import tpu as pltpu
from jax.experimental.pallas import tpu_sc as plsc

NC = 2
NS = 16
LW = 128
F = 16
KROWS = 4
ZROWS = 512

_SC_PARAMS = pltpu.CompilerParams(use_tc_tiling_on_sc=False)


def _sc_geometry(n, r):
    cap = ((n + 256 + NS - 1) // NS) * NS
    stripe_z = cap // NS
    nzc = (stripe_z + ZROWS - 1) // ZROWS
    ws0 = ((n + NS - 1) // NS + 7) // 8 * 8
    ws_last = n - (NS - 1) * ws0
    rs = r // (NC * NS)
    nb = rs // KROWS
    assert n % NS == 0 and r % (NC * NS) == 0 and rs % KROWS == 0
    assert nb % 2 == 0 and stripe_z >= ZROWS
    assert 0 < ws_last <= ws0 and (NS - 1) * ws0 + ws_last == n
    assert cap >= (NS - 1) * ws0 + ws0
    return cap, stripe_z, nzc, ws0, ws_last, rs, nb


def _zero_acc(acc, zsrc, s, stripe_z, nzc):
    z0 = s * stripe_z

    @pl.loop(0, nzc - 1)
    def _(i):
        pltpu.sync_copy(zsrc, acc.at[pl.ds(z0 + i * ZROWS, ZROWS)])

    pltpu.sync_copy(zsrc, acc.at[pl.ds(z0 + stripe_z - ZROWS, ZROWS)])


def _readout(acc, out_hbm, s, ws0, ws_last):
    w0 = pl.multiple_of(s * ws0, 8)

    @pl.when(s < NS - 1)
    def _():
        pltpu.sync_copy(acc.at[pl.ds(w0, ws0)], out_hbm.at[pl.ds(w0, ws0)])

    @pl.when(s == NS - 1)
    def _():
        pltpu.sync_copy(acc.at[pl.ds(w0, ws_last)],
                        out_hbm.at[pl.ds(w0, ws_last)])


def _sc_segsum(n, r, table, edges_r, zeros):
    cap, stripe_z, nzc, ws0, ws_last, rs, nb = _sc_geometry(n, r)
    mesh = plsc.VectorSubcoreMesh(core_axis_name="c", subcore_axis_name="s")
    out_t = jax.ShapeDtypeStruct((n, F), jnp.float32)

    @functools.partial(
        pl.kernel,
        out_type=[out_t, out_t],
        mesh=mesh,
        scratch_types=[
            pltpu.VMEM((KROWS * LW, F), jnp.float32),
            pltpu.VMEM((KROWS * LW, F), jnp.float32),
            pltpu.VMEM((2 * KROWS, LW), jnp.int32),
            pltpu.VMEM((2 * KROWS, LW), jnp.int32),
            pltpu.VMEM_SHARED((cap, F), jnp.float32),
            pltpu.SemaphoreType.DMA,
            pltpu.SemaphoreType.DMA,
            pltpu.SemaphoreType.DMA,
            pltpu.SemaphoreType.DMA,
            pltpu.SemaphoreType.DMA,
            pltpu.SemaphoreType.DMA,
        ],
        compiler_params=_SC_PARAMS,
    )
    def seg_kernel(table_hbm, edges_hbm, zeros_hbm, out0_hbm, out1_hbm,
                   rows0, rows1, ev0, ev1, acc,
                   isem0, isem1, gsem0, gsem1, ssem0, ssem1):
        c = lax.axis_index("c")
        s = lax.axis_index("s")

        pltpu.sync_copy(zeros_hbm, rows0)
        _zero_acc(acc, rows0, s, stripe_z, nzc)
        plsc.subcore_barrier()

        row0 = (c * NS + s) * rs
        bufs = ((rows0, ev0, isem0, gsem0, ssem0),
                (rows1, ev1, isem1, gsem1, ssem1))

        def fire_idx(t, b):
            pltpu.async_copy(
                edges_hbm.at[pl.ds(2 * (row0 + t * KROWS), 2 * KROWS)],
                b[1], b[2])

        def wait_idx(t, b):
            pltpu.make_async_copy(
                edges_hbm.at[pl.ds(2 * (row0 + t * KROWS), 2 * KROWS)],
                b[1], b[2]).wait()

        def fire_gathers(b):
            @pl.loop(0, KROWS)
            def _(j):
                pltpu.async_copy(table_hbm.at[b[1].at[2 * j]],
                                 b[0].at[pl.ds(j * LW, LW)], b[3])

        def drain_gathers(b):
            @pl.loop(0, KROWS)
            def _(j):
                pltpu.make_async_copy(table_hbm.at[b[1].at[2 * j]],
                                      b[0].at[pl.ds(j * LW, LW)], b[3]).wait()

        def fire_scatters(b):
            @pl.loop(0, KROWS)
            def _(j):
                pltpu.async_copy(b[0].at[pl.ds(j * LW, LW)],
                                 acc.at[b[1].at[2 * j + 1]], b[4], add=True)

        def wait_scatters(b):
            @pl.loop(0, KROWS)
            def _(j):
                pltpu.make_async_copy(b[0].at[pl.ds(j * LW, LW)],
                                      acc.at[b[1].at[2 * j + 1]], b[4]).wait()

        def half_step(t, cur, nxt):
            drain_gathers(cur)
            fire_scatters(cur)

            @pl.when(t + 1 < nb)
            def _():
                wait_idx(t + 1, nxt)
                fire_gathers(nxt)

            wait_scatters(cur)

            @pl.when(t + 2 < nb)
            def _():
                fire_idx(t + 2, cur)

        fire_idx(0, bufs[0])
        fire_idx(1, bufs[1])
        wait_idx(0, bufs[0])
        fire_gathers(bufs[0])

        @pl.loop(0, nb // 2)
        def _(tt):
            half_step(2 * tt, bufs[0], bufs[1])
            half_step(2 * tt + 1, bufs[1], bufs[0])

        plsc.subcore_barrier()

        @pl.when(c == 0)
        def _():
            _readout(acc, out0_hbm, s, ws0, ws_last)

        @pl.when(c == 1)
        def _():
            _readout(acc, out1_hbm, s, ws0, ws_last)

    return seg_kernel(table, edges_r, zeros)


def _sc_deg(n, r, edges_r, ones, zeros):
    cap, stripe_z, nzc, ws0, ws_last, rs, nb = _sc_geometry(n, r)
    mesh = plsc.VectorSubcoreMesh(core_axis_name="c", subcore_axis_name="s")
    out_t = jax.ShapeDtypeStruct((n, F), jnp.float32)

    @functools.partial(
        pl.kernel,
        out_type=[out_t, out_t],
        mesh=mesh,
        scratch_types=[
            pltpu.VMEM((ZROWS, F), jnp.float32),
            pltpu.VMEM((LW, F), jnp.float32),
            pltpu.VMEM((2 * KROWS, LW), jnp.int32),
            pltpu.VMEM((2 * KROWS, LW), jnp.int32),
            pltpu.VMEM_SHARED((cap, F), jnp.float32),
            pltpu.SemaphoreType.DMA,
            pltpu.SemaphoreType.DMA,
            pltpu.SemaphoreType.DMA,
            pltpu.SemaphoreType.DMA,
        ],
        compiler_params=_SC_PARAMS,
    )
    def deg_kernel(edges_hbm, ones_hbm, zeros_hbm, out0_hbm, out1_hbm,
                   zero_v, ones_v, ev0, ev1, acc,
                   isem0, isem1, ssem0, ssem1):
        c = lax.axis_index("c")
        s = lax.axis_index("s")

        pltpu.sync_copy(zeros_hbm, zero_v)
        pltpu.sync_copy(ones_hbm, ones_v)
        _zero_acc(acc, zero_v, s, stripe_z, nzc)
        plsc.subcore_barrier()

        row0 = (c * NS + s) * rs
        bufs = ((ev0, isem0, ssem0), (ev1, isem1, ssem1))

        def fire_idx(t, b):
            pltpu.async_copy(
                edges_hbm.at[pl.ds(2 * (row0 + t * KROWS), 2 * KROWS)],
                b[0], b[1])

        def wait_idx(t, b):
            pltpu.make_async_copy(
                edges_hbm.at[pl.ds(2 * (row0 + t * KROWS), 2 * KROWS)],
                b[0], b[1]).wait()

        def fire_scatters(b):
            @pl.loop(0, KROWS)
            def _(j):
                pltpu.async_copy(ones_v, acc.at[b[0].at[2 * j + 1]],
                                 b[2], add=True)

        def wait_scatters(b):
            @pl.loop(0, KROWS)
            def _(j):
                pltpu.make_async_copy(ones_v, acc.at[b[0].at[2 * j + 1]],
                                      b[2]).wait()

        def half_step(t, cur):
            wait_idx(t, cur)
            fire_scatters(cur)
            wait_scatters(cur)

            @pl.when(t + 2 < nb)
            def _():
                fire_idx(t + 2, cur)

        fire_idx(0, bufs[0])
        fire_idx(1, bufs[1])

        @pl.loop(0, nb // 2)
        def _(tt):
            half_step(2 * tt, bufs[0])
            half_step(2 * tt + 1, bufs[1])

        plsc.subcore_barrier()

        @pl.when(c == 0)
        def _():
            _readout(acc, out0_hbm, s, ws0, ws_last)

        @pl.when(c == 1)
        def _():
            _readout(acc, out1_hbm, s, ws0, ws_last)

    return deg_kernel(edges_r, ones, zeros)


G = 25
BP = 500


def _tc_matmul_packed(xp, w1big):
    kp = xp.shape[2]

    def body(x_ref, w_ref, o_ref):
        o_ref[0] = jnp.dot(x_ref[0], w_ref[...],
                           preferred_element_type=jnp.float32,
                           precision=lax.Precision.HIGHEST)

    return pl.pallas_call(
        body,
        grid=(G,),
        in_specs=[pl.BlockSpec((1, BP, kp), lambda i: (i, 0, 0)),
                  pl.BlockSpec((kp, 128), lambda i: (0, 0))],
        out_specs=pl.BlockSpec((1, BP, 128), lambda i: (i, 0, 0)),
        out_shape=jax.ShapeDtypeStruct((G, BP, 128), jnp.float32),
    )(xp, w1big)


_P3SPEC = pl.BlockSpec((1, BP, 128), lambda i: (i, 0, 0))


def _tc_scale(h1p, deg_a, deg_b):
    def body(h_ref, da_ref, db_ref, g_ref, dinv_ref):
        dinv = lax.rsqrt(da_ref[...] + db_ref[...] + 1.0)
        g_ref[...] = h_ref[...] * dinv
        dinv_ref[...] = dinv

    out_t = jax.ShapeDtypeStruct((G, BP, 128), jnp.float32)
    return pl.pallas_call(
        body,
        grid=(G,),
        in_specs=[_P3SPEC, _P3SPEC, _P3SPEC],
        out_specs=[_P3SPEC, _P3SPEC],
        out_shape=[out_t, out_t],
    )(h1p, deg_a, deg_b)


def _tc_mid(seg_a, seg_b, g1p, dinvp, b1tile):
    def body(sa_ref, sb_ref, g_ref, d_ref, b_ref, y_ref):
        dinv_b = d_ref[...]
        z = dinv_b * (sa_ref[...] + sb_ref[...] + g_ref[...]) + b_ref[...]
        z = jnp.maximum(z, 0.0)
        y_ref[...] = z * dinv_b

    return pl.pallas_call(
        body,
        grid=(G,),
        in_specs=[_P3SPEC, _P3SPEC, _P3SPEC, _P3SPEC,
                  pl.BlockSpec((1, 1, 128), lambda i: (0, 0, 0))],
        out_specs=_P3SPEC,
        out_shape=jax.ShapeDtypeStruct((G, BP, 128), jnp.float32),
    )(seg_a, seg_b, g1p, dinvp, b1tile)


def _tc_out_packed(seg_a, seg_b, yp, dinvp, w2big, sel, b2tile,
                   kshrink, kgrow, msum):
    mp = w2big.shape[1]

    def body(sa_ref, sb_ref, y_ref, d_ref, w_ref, s_ref, b_ref,
             ks_ref, kg_ref, ms_ref, o_ref):
        hi = lax.Precision.HIGHEST
        t = sa_ref[0] + sb_ref[0] + y_ref[0]
        h2 = jnp.dot(t, w_ref[...], preferred_element_type=jnp.float32,
                     precision=hi)
        dsc = jnp.dot(d_ref[0], s_ref[...], preferred_element_type=jnp.float32,
                      precision=hi)
        o = dsc * h2 + b_ref[0]
        c1 = jnp.dot(o, ks_ref[...], preferred_element_type=jnp.float32)
        shift = jnp.dot(c1, kg_ref[...], preferred_element_type=jnp.float32)
        oc = o - shift
        e = jnp.exp(oc)
        ssum = jnp.dot(e, ms_ref[...], preferred_element_type=jnp.float32,
                       precision=hi)
        o_ref[0] = oc - jnp.log(ssum)

    return pl.pallas_call(
        body,
        grid=(G,),
        in_specs=[_P3SPEC, _P3SPEC, _P3SPEC, _P3SPEC,
                  pl.BlockSpec((128, mp), lambda i: (0, 0)),
                  pl.BlockSpec((128, mp), lambda i: (0, 0)),
                  pl.BlockSpec((1, 1, mp), lambda i: (0, 0, 0)),
                  pl.BlockSpec((mp, 8), lambda i: (0, 0)),
                  pl.BlockSpec((8, mp), lambda i: (0, 0)),
                  pl.BlockSpec((mp, mp), lambda i: (0, 0))],
        out_specs=pl.BlockSpec((1, BP, mp), lambda i: (i, 0, 0)),
        out_shape=jax.ShapeDtypeStruct((G, BP, mp), jnp.float32),
    )(seg_a, seg_b, yp, dinvp, w2big, sel, b2tile, kshrink, kgrow, msum)


def kernel(x, edge_index, W1, b1, W2, b2):
    n = x.shape[0]
    e = edge_index.shape[1]
    h = W1.shape[1]
    c = W2.shape[1]
    assert h == F

    unit = LW * NC * NS * KROWS * 2
    e_pad = ((e + unit - 1) // unit) * unit
    pad = e_pad - e
    if pad:
        pad_block = jnp.stack(
            [jnp.zeros((pad,), jnp.int32),
             n + (jnp.arange(pad, dtype=jnp.int32) & 255)])
        ei = jnp.concatenate([edge_index, pad_block], axis=1)
    else:
        ei = edge_index
    r = e_pad // LW
    edges_r = ei.reshape(2, r, LW).transpose(1, 0, 2).reshape(2 * r, LW)

    ones = jnp.ones((LW, F), jnp.float32)
    zeros = jnp.zeros((ZROWS, F), jnp.float32)

    def pk(a):
        return a.reshape(G, BP, 8 * F)

    def un(ap):
        return ap.reshape(n, F)

    eye8 = jnp.eye(8, dtype=jnp.float32)
    deg_a, deg_b = _sc_deg(n, r, edges_r, ones, zeros)
    w1big = jnp.kron(eye8, W1)
    h1p = _tc_matmul_packed(x.reshape(G, BP, 8 * x.shape[1]), w1big)
    g1p, dinvp = _tc_scale(h1p, pk(deg_a), pk(deg_b))
    seg1a, seg1b = _sc_segsum(n, r, un(g1p), edges_r, zeros)
    yp = _tc_mid(pk(seg1a), pk(seg1b), g1p, dinvp,
                 jnp.tile(b1, 8).reshape(1, 1, 8 * h))
    seg2a, seg2b = _sc_segsum(n, r, un(yp), edges_r, zeros)
    w2big = jnp.kron(eye8, W2)
    sel = jnp.kron(eye8, jnp.zeros((h, c), jnp.float32).at[0, :].set(1.0))
    b2tile = jnp.tile(b2, 8).reshape(1, 1, 8 * c)
    kshrink = jnp.kron(eye8, jnp.full((c, 1), 1.0 / c, jnp.float32))
    kgrow = jnp.kron(eye8, jnp.ones((1, c), jnp.float32))
    msum = jnp.kron(eye8, jnp.ones((c, c), jnp.float32))
    o_p = _tc_out_packed(pk(seg2a), pk(seg2b), yp, dinvp, w2big, sel, b2tile,
                         kshrink, kgrow, msum)
    return o_p.reshape(n, c)
